# baseline (speedup 1.0000x reference)
import functools

import jax
import jax.numpy as jnp
from jax import lax
from jax.experimental import pallas as pl
from jax.experimental.pallas import tpu as pltpu
from jax.experimental.pallas import tpu_sc as plsc

N_OPS = 8
D_IN = 768
D_OP = 128
ENC = 256
OUT_W = ENC + N_OPS
PAD_W = 384

BLK1 = 512
BLK = 256
NC = 2
NS = 16
NW = NC * NS
NHALF = 2


def _stage1_all_experts(Bn, half):
    nbh = Bn // NHALF // BLK1
    off = half * nbh

    def body(ids_ref, xl_ref, xr_ref, w_ref, b_ref, h_ref):
        xl = xl_ref[...].astype(jnp.bfloat16)
        xr = xr_ref[...].astype(jnp.bfloat16)
        w = w_ref[...]
        H = jnp.dot(xl, w[:D_IN // 2], preferred_element_type=jnp.float32)
        H = H + jnp.dot(xr, w[D_IN // 2:], preferred_element_type=jnp.float32)
        H = H + b_ref[...]
        ids = ids_ref[...]
        acc = jnp.zeros((BLK1, D_OP), jnp.float32)
        for k in range(N_OPS):
            acc = jnp.where(ids == k, H[:, k * D_OP:(k + 1) * D_OP], acc)
        h_ref[...] = acc

    return pl.pallas_call(
        body,
        grid=(nbh,),
        in_specs=[
            pl.BlockSpec((BLK1, 1), lambda i: (i + off, 0)),
            pl.BlockSpec((BLK1, D_IN // 2), lambda i: (i + off, 0)),
            pl.BlockSpec((BLK1, D_IN // 2), lambda i: (i + off, 1)),
            pl.BlockSpec(memory_space=pltpu.VMEM),
            pl.BlockSpec(memory_space=pltpu.VMEM),
        ],
        out_specs=pl.BlockSpec((BLK1, D_OP), lambda i: (i, 0)),
        out_shape=jax.ShapeDtypeStruct((Bn // NHALF, D_OP), jnp.float32),
    )


def _scatter_rows(n_src, n_dst, n_cols, chunk):
    per_w = n_src // NW
    n_chunks = per_w // chunk
    mesh = plsc.VectorSubcoreMesh(core_axis_name="c", subcore_axis_name="s",
                                  num_cores=NC, num_subcores=NS)

    @functools.partial(
        pl.kernel,
        out_type=jax.ShapeDtypeStruct((n_dst, n_cols), jnp.float32),
        mesh=mesh,
        scratch_types=[
            pltpu.VMEM((2, chunk), jnp.int32),
            pltpu.VMEM((2, chunk, n_cols), jnp.float32),
            pltpu.SemaphoreType.DMA((2,)),
            pltpu.SemaphoreType.DMA((2,)),
        ],
    )
    def scatter(idx_hbm, src_hbm, dst_hbm, idx_v, rows_v, rsem, wsem):
        wid = lax.axis_index("s") * NC + lax.axis_index("c")
        base = wid * per_w

        def start_read(c):
            b = c % 2
            pltpu.sync_copy(idx_hbm.at[pl.ds(base + c * chunk, chunk)],
                            idx_v.at[b])
            return pltpu.async_copy(
                src_hbm.at[pl.ds(base + c * chunk, chunk)],
                rows_v.at[b], rsem.at[b])

        r = start_read(0)
        pending_wr = [None, None]
        for c in range(n_chunks):
            b = c % 2
            r.wait()
            if c + 1 < n_chunks:
                b1 = (c + 1) % 2
                if pending_wr[b1] is not None:
                    pending_wr[b1].wait()
                    pending_wr[b1] = None
                r = start_read(c + 1)
            pending_wr[b] = pltpu.async_copy(
                rows_v.at[b], dst_hbm.at[idx_v.at[b]], wsem.at[b])
        for wr in pending_wr:
            if wr is not None:
                wr.wait()

    return scatter


def _gather2(n_rows, chunk):
    half = n_rows // NHALF
    per_w = half // NW
    n_chunks = per_w // chunk
    mesh = plsc.VectorSubcoreMesh(core_axis_name="c", subcore_axis_name="s",
                                  num_cores=NC, num_subcores=NS)

    @functools.partial(
        pl.kernel,
        out_type=jax.ShapeDtypeStruct((n_rows, PAD_W), jnp.float32),
        mesh=mesh,
        scratch_types=[
            pltpu.VMEM((per_w,), jnp.int32),
            pltpu.VMEM((2, chunk, PAD_W), jnp.float32),
            pltpu.SemaphoreType.DMA((2,)),
            pltpu.SemaphoreType.DMA((2,)),
        ],
    )
    def gather(idx_hbm, ya_hbm, yb_hbm, out_hbm, idx_v, rows_v, gsem, wsem):
        wid = lax.axis_index("s") * NC + lax.axis_index("c")

        for p, src_hbm in enumerate((ya_hbm, yb_hbm)):
            base = p * half + wid * per_w
            pltpu.sync_copy(idx_hbm.at[pl.ds(base, per_w)], idx_v)

            def start_gather(c):
                b = c % 2
                return pltpu.async_copy(
                    src_hbm.at[idx_v.at[pl.ds(c * chunk, chunk)]],
                    rows_v.at[b], gsem.at[b])

            g = start_gather(0)
            pending_wb = [None, None]
            for c in range(n_chunks):
                b = c % 2
                g.wait()
                if c + 1 < n_chunks:
                    b1 = (c + 1) % 2
                    if pending_wb[b1] is not None:
                        pending_wb[b1].wait()
                        pending_wb[b1] = None
                    g = start_gather(c + 1)
                pending_wb[b] = pltpu.async_copy(
                    rows_v.at[b], out_hbm.at[pl.ds(base + c * chunk, chunk)],
                    wsem.at[b])
            for wb in pending_wb:
                if wb is not None:
                    wb.wait()

    return gather


def _stage2_expert(n_rows):
    nb = n_rows // BLK

    def body(be_ref, h_ref, w2_ref, b2_ref, o_ref):
        k = be_ref[pl.program_id(0)]
        y = jnp.dot(h_ref[...], w2_ref[k], preferred_element_type=jnp.float32)
        y = y + b2_ref[k]
        ind = (lax.broadcasted_iota(jnp.int32, (BLK, PAD_W - ENC), 1) == k)
        o_ref[...] = jnp.concatenate([y, ind.astype(jnp.float32)], axis=1)

    return pl.pallas_call(
        body,
        grid=(nb,),
        in_specs=[
            pl.BlockSpec(memory_space=pltpu.SMEM),
            pl.BlockSpec((BLK, D_OP), lambda i: (i, 0)),
            pl.BlockSpec(memory_space=pltpu.VMEM),
            pl.BlockSpec(memory_space=pltpu.VMEM),
        ],
        out_specs=pl.BlockSpec((BLK, PAD_W), lambda i: (i, 0)),
        out_shape=jax.ShapeDtypeStruct((n_rows, PAD_W), jnp.float32),
    )


def kernel(x, op_ids, op_W, op_b, Ws_W, Ws_b):
    B = x.shape[0]
    B2 = B // NHALF
    Sh = B2 + N_OPS * BLK

    ids = op_ids.astype(jnp.int32)
    ids2 = ids.reshape(NHALF, B2)
    oh32 = (ids2[:, :, None] == jnp.arange(N_OPS, dtype=jnp.int32)[None, None, :]
            ).astype(jnp.int32)
    csum = jnp.cumsum(oh32, axis=1)
    rank = jnp.sum(csum * oh32, axis=2) - 1
    counts = csum[:, -1, :]
    padded = ((counts + BLK - 1) // BLK) * BLK
    starts = jnp.concatenate(
        [jnp.zeros((NHALF, 1), jnp.int32),
         jnp.cumsum(padded, axis=1)[:, :-1].astype(jnp.int32)], axis=1)
    slot = jnp.sum(starts[:, None, :] * oh32, axis=2) + rank
    nbh = Sh // BLK

    bes = []
    for p in range(NHALF):
        bes.append((jnp.searchsorted(
            starts[p], jnp.arange(nbh, dtype=jnp.int32) * BLK,
            side="right") - 1).astype(jnp.int32))

    W_cat = jnp.transpose(op_W, (1, 0, 2)).reshape(D_IN, N_OPS * D_OP)
    W_cat = W_cat.astype(jnp.bfloat16)
    b_cat = op_b.reshape(1, N_OPS * D_OP)
    b2r = Ws_b.reshape(N_OPS, 1, ENC)
    ids_col = ids.reshape(B, 1)

    ys = []
    for p in range(NHALF):
        h_p = _stage1_all_experts(B, p)(ids_col, x, x, W_cat, b_cat)
        hs_p = _scatter_rows(B2, Sh, D_OP, 128)(slot[p], h_p)
        ys.append(_stage2_expert(Sh)(bes[p], hs_p, Ws_W, b2r))

    slot_flat = slot.reshape(B)
    out_full = _gather2(B, 128)(slot_flat, ys[0], ys[1])
    return out_full[:, :OUT_W]


# speedup vs baseline: 1.3228x; 1.3228x over previous
import functools

import jax
import jax.numpy as jnp
from jax import lax
from jax.experimental import pallas as pl
from jax.experimental.pallas import tpu as pltpu
from jax.experimental.pallas import tpu_sc as plsc

N_OPS = 8
D_IN = 768
D_OP = 128
ENC = 256
OUT_W = ENC + N_OPS
PAD_W = 384

BLK1 = 512
BLK = 512
NC = 2
NS = 16
NW = NC * NS
NHALF = 2


def _stage1_all_experts(Bn, half):
    nbh = Bn // NHALF // BLK1
    off = half * nbh

    def body(ids_ref, x_ref, w_ref, b_ref, h_ref):
        xb = x_ref[...].astype(jnp.bfloat16)
        H = jnp.dot(xb, w_ref[...], preferred_element_type=jnp.float32)
        H = H + b_ref[...]
        ids = ids_ref[...]
        acc = jnp.zeros((BLK1, D_OP), jnp.float32)
        for k in range(N_OPS):
            acc = jnp.where(ids == k, H[:, k * D_OP:(k + 1) * D_OP], acc)
        h_ref[...] = acc

    return pl.pallas_call(
        body,
        grid=(nbh,),
        in_specs=[
            pl.BlockSpec((BLK1, 1), lambda i: (i + off, 0)),
            pl.BlockSpec((BLK1, D_IN), lambda i: (i + off, 0)),
            pl.BlockSpec(memory_space=pltpu.VMEM),
            pl.BlockSpec(memory_space=pltpu.VMEM),
        ],
        out_specs=pl.BlockSpec((BLK1, D_OP), lambda i: (i, 0)),
        out_shape=jax.ShapeDtypeStruct((Bn // NHALF, D_OP), jnp.float32),
    )


def _scatter_rows(n_src, n_dst, n_cols, chunk):
    per_w = n_src // NW
    n_chunks = per_w // chunk
    mesh = plsc.VectorSubcoreMesh(core_axis_name="c", subcore_axis_name="s",
                                  num_cores=NC, num_subcores=NS)

    @functools.partial(
        pl.kernel,
        out_type=jax.ShapeDtypeStruct((n_dst, n_cols), jnp.float32),
        mesh=mesh,
        scratch_types=[
            pltpu.VMEM((2, chunk), jnp.int32),
            pltpu.VMEM((2, chunk, n_cols), jnp.float32),
            pltpu.SemaphoreType.DMA((2,)),
            pltpu.SemaphoreType.DMA((2,)),
        ],
    )
    def scatter(idx_hbm, src_hbm, dst_hbm, idx_v, rows_v, rsem, wsem):
        wid = lax.axis_index("s") * NC + lax.axis_index("c")
        base = wid * per_w

        def start_read(c):
            b = c % 2
            pltpu.sync_copy(idx_hbm.at[pl.ds(base + c * chunk, chunk)],
                            idx_v.at[b])
            return pltpu.async_copy(
                src_hbm.at[pl.ds(base + c * chunk, chunk)],
                rows_v.at[b], rsem.at[b])

        r = start_read(0)
        pending_wr = [None, None]
        for c in range(n_chunks):
            b = c % 2
            r.wait()
            if c + 1 < n_chunks:
                b1 = (c + 1) % 2
                if pending_wr[b1] is not None:
                    pending_wr[b1].wait()
                    pending_wr[b1] = None
                r = start_read(c + 1)
            pending_wr[b] = pltpu.async_copy(
                rows_v.at[b], dst_hbm.at[idx_v.at[b]], wsem.at[b])
        for wr in pending_wr:
            if wr is not None:
                wr.wait()

    return scatter


def _gather2(n_rows, chunk):
    half = n_rows // NHALF
    per_w = half // NW
    n_chunks = per_w // chunk
    mesh = plsc.VectorSubcoreMesh(core_axis_name="c", subcore_axis_name="s",
                                  num_cores=NC, num_subcores=NS)

    @functools.partial(
        pl.kernel,
        out_type=jax.ShapeDtypeStruct((n_rows, PAD_W), jnp.float32),
        mesh=mesh,
        scratch_types=[
            pltpu.VMEM((per_w,), jnp.int32),
            pltpu.VMEM((2, chunk, PAD_W), jnp.float32),
            pltpu.SemaphoreType.DMA((2,)),
            pltpu.SemaphoreType.DMA((2,)),
        ],
    )
    def gather(idx_hbm, ya_hbm, yb_hbm, out_hbm, idx_v, rows_v, gsem, wsem):
        wid = lax.axis_index("s") * NC + lax.axis_index("c")

        for p, src_hbm in enumerate((ya_hbm, yb_hbm)):
            base = p * half + wid * per_w
            pltpu.sync_copy(idx_hbm.at[pl.ds(base, per_w)], idx_v)

            def start_gather(c):
                b = c % 2
                return pltpu.async_copy(
                    src_hbm.at[idx_v.at[pl.ds(c * chunk, chunk)]],
                    rows_v.at[b], gsem.at[b])

            g = start_gather(0)
            pending_wb = [None, None]
            for c in range(n_chunks):
                b = c % 2
                g.wait()
                if c + 1 < n_chunks:
                    b1 = (c + 1) % 2
                    if pending_wb[b1] is not None:
                        pending_wb[b1].wait()
                        pending_wb[b1] = None
                    g = start_gather(c + 1)
                pending_wb[b] = pltpu.async_copy(
                    rows_v.at[b], out_hbm.at[pl.ds(base + c * chunk, chunk)],
                    wsem.at[b])
            for wb in pending_wb:
                if wb is not None:
                    wb.wait()

    return gather


def _stage2_expert(n_rows):
    nb = n_rows // BLK

    def body(be_ref, h_ref, w2_ref, b2_ref, o_ref):
        k = be_ref[pl.program_id(0)]
        y = jnp.dot(h_ref[...], w2_ref[k], preferred_element_type=jnp.float32)
        y = y + b2_ref[k]
        ind = (lax.broadcasted_iota(jnp.int32, (BLK, PAD_W - ENC), 1) == k)
        o_ref[...] = jnp.concatenate([y, ind.astype(jnp.float32)], axis=1)

    return pl.pallas_call(
        body,
        grid=(nb,),
        in_specs=[
            pl.BlockSpec(memory_space=pltpu.SMEM),
            pl.BlockSpec((BLK, D_OP), lambda i: (i, 0)),
            pl.BlockSpec(memory_space=pltpu.VMEM),
            pl.BlockSpec(memory_space=pltpu.VMEM),
        ],
        out_specs=pl.BlockSpec((BLK, PAD_W), lambda i: (i, 0)),
        out_shape=jax.ShapeDtypeStruct((n_rows, PAD_W), jnp.float32),
    )


def kernel(x, op_ids, op_W, op_b, Ws_W, Ws_b):
    B = x.shape[0]
    B2 = B // NHALF
    Sh = B2 + N_OPS * BLK

    ids = op_ids.astype(jnp.int32)
    ids2 = ids.reshape(NHALF, B2)
    oh32 = (ids2[:, :, None] == jnp.arange(N_OPS, dtype=jnp.int32)[None, None, :]
            ).astype(jnp.int32)
    csum = jnp.cumsum(oh32, axis=1)
    rank = jnp.sum(csum * oh32, axis=2) - 1
    counts = csum[:, -1, :]
    padded = ((counts + BLK - 1) // BLK) * BLK
    starts = jnp.concatenate(
        [jnp.zeros((NHALF, 1), jnp.int32),
         jnp.cumsum(padded, axis=1)[:, :-1].astype(jnp.int32)], axis=1)
    slot = jnp.sum(starts[:, None, :] * oh32, axis=2) + rank
    nbh = Sh // BLK

    blk_pos = jnp.arange(nbh, dtype=jnp.int32)[:, None] * BLK
    bes = []
    for p in range(NHALF):
        bes.append(jnp.sum(
            (starts[p][None, :] <= blk_pos).astype(jnp.int32), axis=1) - 1)

    W_cat = jnp.transpose(op_W, (1, 0, 2)).reshape(D_IN, N_OPS * D_OP)
    W_cat = W_cat.astype(jnp.bfloat16)
    b_cat = op_b.reshape(1, N_OPS * D_OP)
    b2r = Ws_b.reshape(N_OPS, 1, ENC)
    ids_col = ids.reshape(B, 1)

    ys = []
    for p in range(NHALF):
        h_p = _stage1_all_experts(B, p)(ids_col, x, W_cat, b_cat)
        hs_p = _scatter_rows(B2, Sh, D_OP, 128)(slot[p], h_p)
        ys.append(_stage2_expert(Sh)(bes[p], hs_p, Ws_W, b2r))

    slot_flat = slot.reshape(B)
    out_full = _gather2(B, 128)(slot_flat, ys[0], ys[1])
    return out_full[:, :OUT_W]
